# baseline (speedup 1.0000x reference)
import functools

import jax
import jax.numpy as jnp
from jax import lax
from jax.experimental import pallas as pl
from jax.experimental.pallas import tpu as pltpu
from jax.experimental.pallas import tpu_sc as plsc

_BETA = 0.25
_NE = 8192
_DIM = 256
_NROWS = 9216
_NHALF = _NROWS // 2
_BR = 512
_NBLK = _NHALF // _BR

_NW = 32
_BPW = _NHALF // _NW
_CH = 72
_NCH = _BPW // _CH


def _argmin_body(z_ref, et_ref, en_ref, idx_ref, dsum_ref):
    i = pl.program_id(0)
    z_blk = z_ref[...]
    s = jnp.dot(z_blk, et_ref[...],
                preferred_element_type=jnp.float32)
    zn = jnp.sum(z_blk * z_blk, axis=1, keepdims=True)
    d = (zn + en_ref[...]) - 2.0 * s
    m = jnp.min(d, axis=1, keepdims=True)
    ji = lax.broadcasted_iota(jnp.int32, d.shape, 1)
    idx = jnp.min(jnp.where(d == m, ji, _NE), axis=1)
    idx_ref[pl.ds(i, 1), :] = idx.reshape(1, _BR)

    @pl.when(i == 0)
    def _init():
        dsum_ref[0, 0] = 0.0

    dsum_ref[0, 0] += jnp.sum(m[:, 0])


_dist_argmin = pl.pallas_call(
    _argmin_body,
    grid=(_NBLK,),
    in_specs=[
        pl.BlockSpec((_BR, _DIM), lambda i: (i, 0)),
        pl.BlockSpec((_DIM, _NE), lambda i: (0, 0)),
        pl.BlockSpec((1, _NE), lambda i: (0, 0)),
    ],
    out_specs=[
        pl.BlockSpec((_NBLK, _BR), lambda i: (0, 0)),
        pl.BlockSpec(memory_space=pltpu.SMEM),
    ],
    out_shape=[
        jax.ShapeDtypeStruct((_NBLK, _BR), jnp.int32),
        jax.ShapeDtypeStruct((1, 1), jnp.float32),
    ],
)


_sc_mesh = plsc.VectorSubcoreMesh(core_axis_name="c", subcore_axis_name="s")


@functools.partial(
    pl.kernel,
    mesh=_sc_mesh,
    out_type=jax.ShapeDtypeStruct((_NHALF, _DIM), jnp.float32),
    scratch_types=[
        pltpu.VMEM((_NCH, _CH), jnp.int32),
        pltpu.VMEM((_BPW, _DIM), jnp.float32),
        pltpu.SemaphoreType.DMA,
    ],
)
def _gather_rows(table_hbm, idx_hbm, out_hbm, idx_v, rows_v, sem):
    wid = lax.axis_index("s") * 2 + lax.axis_index("c")
    base = wid * _BPW
    for j in range(_NCH):
        pltpu.sync_copy(idx_hbm.at[pl.ds(base + j * _CH, _CH)], idx_v.at[j])
    copies = [
        pltpu.async_copy(table_hbm.at[idx_v.at[j]],
                         rows_v.at[pl.ds(j * _CH, _CH)], sem)
        for j in range(_NCH)
    ]
    for c in copies:
        c.wait()
    pltpu.sync_copy(rows_v, out_hbm.at[pl.ds(base, _BPW)])


def kernel(z, embedding):
    z_flat = z.reshape(-1, _DIM)
    e_t = embedding.T
    en = jnp.sum(embedding ** 2, axis=1).reshape(1, _NE)
    idx2d_a, dsum_a = _dist_argmin(z_flat[:_NHALF], e_t, en)
    idx_a = idx2d_a.reshape(-1)
    zq_a = _gather_rows(embedding, idx_a)
    idx2d_b, dsum_b = _dist_argmin(z_flat[_NHALF:], e_t, en)
    idx_b = idx2d_b.reshape(-1)
    zq_b = _gather_rows(embedding, idx_b)
    idx = jnp.concatenate([idx_a, idx_b])
    zq_flat = jnp.concatenate([zq_a, zq_b], axis=0)
    z_q = zq_flat.reshape(z.shape)
    z_q_out = jnp.transpose(z_q, (0, 3, 1, 2))
    diff = (dsum_a[0, 0] + dsum_b[0, 0]) * (_BETA / z.size)
    perplexity = jnp.float32(1.0)
    return (z_q_out, diff, idx, perplexity)


# speedup vs baseline: 1.1296x; 1.1296x over previous
import functools

import jax
import jax.numpy as jnp
from jax import lax
from jax.experimental import pallas as pl
from jax.experimental.pallas import tpu as pltpu
from jax.experimental.pallas import tpu_sc as plsc

_BETA = 0.25
_NE = 8192
_DIM = 256
_NROWS = 9216
_BR = 1024
_NBLK = _NROWS // _BR

_NW = 32
_BPW = _NROWS // _NW
_CH = 96
_NCH = _BPW // _CH


def _argmin_body(z_ref, et_ref, en_ref, idx_ref, dsum_ref):
    i = pl.program_id(0)
    z_blk = z_ref[...]
    s = jnp.dot(z_blk, et_ref[...],
                preferred_element_type=jnp.float32)
    zn = jnp.sum(z_blk * z_blk, axis=1, keepdims=True)
    d = (zn + en_ref[...]) - 2.0 * s
    m = jnp.min(d, axis=1, keepdims=True)
    ji = lax.broadcasted_iota(jnp.int32, d.shape, 1)
    idx = jnp.min(jnp.where(d == m, ji, _NE), axis=1)
    idx_ref[pl.ds(i, 1), :] = idx.reshape(1, _BR)

    @pl.when(i == 0)
    def _init():
        dsum_ref[0, 0] = 0.0

    dsum_ref[0, 0] += jnp.sum(m[:, 0])


_dist_argmin = pl.pallas_call(
    _argmin_body,
    grid=(_NBLK,),
    in_specs=[
        pl.BlockSpec((_BR, _DIM), lambda i: (i, 0)),
        pl.BlockSpec((_DIM, _NE), lambda i: (0, 0)),
        pl.BlockSpec((1, _NE), lambda i: (0, 0)),
    ],
    out_specs=[
        pl.BlockSpec((_NBLK, _BR), lambda i: (0, 0)),
        pl.BlockSpec(memory_space=pltpu.SMEM),
    ],
    out_shape=[
        jax.ShapeDtypeStruct((_NBLK, _BR), jnp.int32),
        jax.ShapeDtypeStruct((1, 1), jnp.float32),
    ],
)


_sc_mesh = plsc.VectorSubcoreMesh(core_axis_name="c", subcore_axis_name="s")


@functools.partial(
    pl.kernel,
    mesh=_sc_mesh,
    out_type=jax.ShapeDtypeStruct((_NROWS, _DIM), jnp.float32),
    scratch_types=[
        pltpu.VMEM((_NCH, _CH), jnp.int32),
        pltpu.VMEM((_BPW, _DIM), jnp.float32),
        pltpu.SemaphoreType.DMA,
    ],
)
def _gather_rows(table_hbm, idx_hbm, out_hbm, idx_v, rows_v, sem):
    wid = lax.axis_index("s") * 2 + lax.axis_index("c")
    base = wid * _BPW
    for j in range(_NCH):
        pltpu.sync_copy(idx_hbm.at[pl.ds(base + j * _CH, _CH)], idx_v.at[j])
    copies = [
        pltpu.async_copy(table_hbm.at[idx_v.at[j]],
                         rows_v.at[pl.ds(j * _CH, _CH)], sem)
        for j in range(_NCH)
    ]
    for c in copies:
        c.wait()
    pltpu.sync_copy(rows_v, out_hbm.at[pl.ds(base, _BPW)])


def kernel(z, embedding):
    z_flat = z.reshape(-1, _DIM)
    e_t = embedding.T
    en = jnp.sum(embedding ** 2, axis=1).reshape(1, _NE)
    idx2d, dsum = _dist_argmin(z_flat, e_t, en)
    idx = idx2d.reshape(-1)
    zq_flat = _gather_rows(embedding, idx)
    z_q = zq_flat.reshape(z.shape)
    z_q_out = jnp.transpose(z_q, (0, 3, 1, 2))
    diff = dsum[0, 0] * (_BETA / z.size)
    perplexity = jnp.float32(1.0)
    return (z_q_out, diff, idx, perplexity)
